# Initial kernel scaffold; baseline (speedup 1.0000x reference)
import jax
import jax.numpy as jnp
from jax.experimental import pallas as pl


def kernel(x, weight):
    raise NotImplementedError("write your pallas kernel here")


# speedup vs baseline: 2.2982x; 2.2982x over previous
import functools

import jax
import jax.numpy as jnp
from jax.experimental import pallas as pl
from jax.experimental.pallas import tpu as pltpu


def _matmul_nt_kernel(x_ref, w_ref, o_ref):
    o_ref[...] = jax.lax.dot_general(
        x_ref[...],
        w_ref[...],
        dimension_numbers=(((1,), (1,)), ((), ())),
        preferred_element_type=jnp.float32,
    )


def _round_up(v: int, m: int) -> int:
    return -(-v // m) * m


@functools.partial(jax.jit, static_argnames=("bm", "bn"))
def _linear_no_bias(x, weight, *, bm=1024, bn=1024):
    B, K = x.shape
    N, K2 = weight.shape
    assert K == K2, "in_features mismatch"

    xb = x.astype(jnp.bfloat16)
    wb = weight.astype(jnp.bfloat16)

    bm = min(bm, _round_up(B, 16))
    bn = min(bn, _round_up(N, 128))
    Bp, Np, Kp = _round_up(B, bm), _round_up(N, bn), _round_up(K, 128)
    if Bp != B or Kp != K:
        xb = jnp.pad(xb, ((0, Bp - B), (0, Kp - K)))
    if Np != N or Kp != K:
        wb = jnp.pad(wb, ((0, Np - N), (0, Kp - K)))

    out = pl.pallas_call(
        _matmul_nt_kernel,
        out_shape=jax.ShapeDtypeStruct((Bp, Np), jnp.float32),
        grid=(Bp // bm, Np // bn),
        in_specs=[
            pl.BlockSpec((bm, Kp), lambda i, j: (i, 0)),
            pl.BlockSpec((bn, Kp), lambda i, j: (j, 0)),
        ],
        out_specs=pl.BlockSpec((bm, bn), lambda i, j: (i, j)),
        compiler_params=pltpu.CompilerParams(
            dimension_semantics=("parallel", "parallel"),
        ),
        cost_estimate=pl.CostEstimate(
            flops=2 * B * N * K,
            transcendentals=0,
            bytes_accessed=(B * K + K * N) * 2 + B * N * 4,
        ),
    )(xb, wb)

    if Bp != B or Np != N:
        out = out[:B, :N]
    return out


def kernel(x, weight):
    return _linear_no_bias(x, weight)


# speedup vs baseline: 2.6824x; 1.1672x over previous
import functools

import jax
import jax.numpy as jnp
from jax.experimental import pallas as pl
from jax.experimental.pallas import tpu as pltpu


def _matmul_nt_kernel(x_ref, w_ref, o_ref):
    o_ref[...] = jax.lax.dot_general(
        x_ref[...].astype(jnp.bfloat16),
        w_ref[...],
        dimension_numbers=(((1,), (1,)), ((), ())),
        preferred_element_type=jnp.float32,
    )


def _round_up(v: int, m: int) -> int:
    return -(-v // m) * m


@functools.partial(jax.jit, static_argnames=("bm",))
def _linear_no_bias(x, weight, *, bm=256):
    B, K = x.shape
    N, K2 = weight.shape
    assert K == K2, "in_features mismatch"

    wb = weight.astype(jnp.bfloat16)

    bm = min(bm, _round_up(B, 16))
    Bp, Np, Kp = _round_up(B, bm), _round_up(N, 128), _round_up(K, 128)
    if Bp != B or Kp != K:
        x = jnp.pad(x, ((0, Bp - B), (0, Kp - K)))
    if Np != N or Kp != K:
        wb = jnp.pad(wb, ((0, Np - N), (0, Kp - K)))

    out = pl.pallas_call(
        _matmul_nt_kernel,
        out_shape=jax.ShapeDtypeStruct((Bp, Np), jnp.float32),
        grid=(Bp // bm,),
        in_specs=[
            pl.BlockSpec((bm, Kp), lambda i: (i, 0)),
            pl.BlockSpec((Np, Kp), lambda i: (0, 0)),
        ],
        out_specs=pl.BlockSpec((bm, Np), lambda i: (i, 0)),
        compiler_params=pltpu.CompilerParams(
            dimension_semantics=("parallel",),
        ),
        cost_estimate=pl.CostEstimate(
            flops=2 * B * N * K,
            transcendentals=0,
            bytes_accessed=B * K * 4 + K * N * 2 + B * N * 4,
        ),
    )(x, wb)

    if Bp != B or Np != N:
        out = out[:B, :N]
    return out


def kernel(x, weight):
    return _linear_no_bias(x, weight)
